# baseline (speedup 1.0000x reference)
import functools
import math

import jax
import jax.numpy as jnp
from jax import lax
from jax.experimental import pallas as pl
from jax.experimental.pallas import tpu as pltpu
from jax.experimental.pallas import tpu_sc as plsc

_T = 4096
_D = 1024
_E = 8
_CAPF = 1.25
_MINCAP = 4


def _capacity(num_tokens, num_experts):
    cap = math.floor(_CAPF * num_tokens / num_experts)
    cap += cap % 2
    return max(cap, _MINCAP)


def _router_body(x_ref, w_ref, gv_ref, dest_ref, cap):
    x = x_ref[...]
    w = w_ref[...]
    logits = lax.dot_general(x, w, (((1,), (1,)), ((), ())),
                             preferred_element_type=jnp.float32)
    m = jnp.max(logits, axis=1, keepdims=True)
    denom = jnp.sum(jnp.exp(logits - m), axis=1)
    gv = 1.0 / denom
    top1 = jnp.argmax(logits, axis=1).astype(jnp.int32)
    onehot = (lax.broadcasted_iota(jnp.int32, logits.shape, 1)
              == top1[:, None])
    counts = onehot.astype(jnp.int32)
    off = 1
    while off < counts.shape[0]:
        shifted = jnp.concatenate(
            [jnp.zeros((off, counts.shape[1]), counts.dtype), counts[:-off]],
            axis=0)
        counts = counts + shifted
        off *= 2
    rank = jnp.sum(jnp.where(onehot, counts - 1, 0), axis=1)
    kept = rank < cap
    dest = jnp.where(kept, top1 * cap + rank, -1)
    gv_ref[...] = gv
    dest_ref[...] = dest


def _slotmap_body(dest_ref, gv_ref, gidx_ref, gvs_ref, nslot):
    def init(s, _):
        gidx_ref[s] = 0
        gvs_ref[s] = 0.0
        return 0

    lax.fori_loop(0, nslot, init, 0)

    def fill(t, _):
        d = dest_ref[t]

        @pl.when(d >= 0)
        def _():
            gidx_ref[d] = t
            gvs_ref[d] = gv_ref[t]

        return 0

    lax.fori_loop(0, _T, fill, 0)


def _sc_dispatch_body(gidx_hbm, gvw_hbm, x_hbm, out_hbm,
                      idx_v, gvw_v, rows_v, sem, rows_per_w, half_rows):
    nc = 2
    wid = lax.axis_index("s") * nc + lax.axis_index("c")
    base = wid * rows_per_w
    pltpu.sync_copy(gidx_hbm.at[pl.ds(base, rows_per_w)], idx_v)
    for half in range(rows_per_w // half_rows):
        off = half * half_rows
        pltpu.sync_copy(gvw_hbm.at[pl.ds(base + off, half_rows)], gvw_v)
        pltpu.async_copy(
            x_hbm.at[idx_v.at[pl.ds(off, half_rows)]], rows_v, sem).wait()

        def scale(r, _):
            gvr = gvw_v[r, :]
            for j in range(_D // 16):
                sl = pl.ds(j * 16, 16)
                rows_v[r, sl] = rows_v[r, sl] * gvr
            return 0

        lax.fori_loop(0, half_rows, scale, 0)
        pltpu.sync_copy(rows_v, out_hbm.at[pl.ds(base + off, half_rows)])


def _combine_body(code_ref, gv_ref, comb_ref, sec_ref):
    code = code_ref[...]
    gv = gv_ref[...]
    slot = lax.broadcasted_iota(jnp.int32, comb_ref.shape, 1)
    hit = slot == code
    comb_ref[...] = jnp.where(hit, gv, 0.0)
    sec_ref[...] = hit


def kernel(x, gate_weight):
    cap = _capacity(_T, _E)
    ec = _E * cap

    router = pl.pallas_call(
        functools.partial(_router_body, cap=cap),
        out_shape=(
            jax.ShapeDtypeStruct((_T,), jnp.float32),
            jax.ShapeDtypeStruct((_T,), jnp.int32),
        ),
    )
    gv, dest = router(x, gate_weight)

    slotmap = pl.pallas_call(
        functools.partial(_slotmap_body, nslot=ec),
        in_specs=[
            pl.BlockSpec(memory_space=pltpu.SMEM),
            pl.BlockSpec(memory_space=pltpu.SMEM),
        ],
        out_specs=(
            pl.BlockSpec(memory_space=pltpu.SMEM),
            pl.BlockSpec(memory_space=pltpu.SMEM),
        ),
        out_shape=(
            jax.ShapeDtypeStruct((ec,), jnp.int32),
            jax.ShapeDtypeStruct((ec,), jnp.float32),
        ),
    )
    gidx, gvs = slotmap(dest, gv)

    nw = 32
    rows_per_w = ec // nw
    half_rows = 80
    mesh = plsc.VectorSubcoreMesh(core_axis_name="c", subcore_axis_name="s")
    sc_dispatch = functools.partial(
        pl.kernel,
        mesh=mesh,
        out_type=jax.ShapeDtypeStruct((ec, _D), jnp.float32),
        scratch_types=[
            pltpu.VMEM((rows_per_w,), jnp.int32),
            pltpu.VMEM((half_rows, 16), jnp.float32),
            pltpu.VMEM((half_rows, _D), jnp.float32),
            pltpu.SemaphoreType.DMA,
        ],
    )(functools.partial(_sc_dispatch_body,
                        rows_per_w=rows_per_w, half_rows=half_rows))
    gvw = jnp.broadcast_to(gvs[:, None], (ec, 16))
    dispatched = sc_dispatch(gidx, gvw, x).reshape(_E, cap, _D)

    tb = 256
    nblk = _T // tb
    meta_spec = pl.BlockSpec((tb, 1), lambda i: (i, 0))
    combine = pl.pallas_call(
        _combine_body,
        grid=(nblk,),
        in_specs=[meta_spec, meta_spec],
        out_specs=(
            pl.BlockSpec((tb, ec), lambda i: (i, 0)),
            pl.BlockSpec((tb, ec), lambda i: (i, 0)),
        ),
        out_shape=(
            jax.ShapeDtypeStruct((_T, ec), jnp.float32),
            jax.ShapeDtypeStruct((_T, ec), jnp.bool_),
        ),
    )
    comb, sec = combine(dest.reshape(_T, 1), gv.reshape(_T, 1))
    return (dispatched,
            comb.reshape(_T, _E, cap),
            sec.reshape(_T, _E, cap))


# speedup vs baseline: 1.0848x; 1.0848x over previous
import functools
import math

import jax
import jax.numpy as jnp
from jax import lax
from jax.experimental import pallas as pl
from jax.experimental.pallas import tpu as pltpu
from jax.experimental.pallas import tpu_sc as plsc

_T = 4096
_D = 1024
_E = 8
_CAPF = 1.25
_MINCAP = 4


def _capacity(num_tokens, num_experts):
    cap = math.floor(_CAPF * num_tokens / num_experts)
    cap += cap % 2
    return max(cap, _MINCAP)


def _router_body(x_ref, w_ref, gv_ref, dest_ref, y_ref, cap):
    x = x_ref[...]
    w = w_ref[...]
    logits = lax.dot_general(x, w, (((1,), (1,)), ((), ())),
                             preferred_element_type=jnp.float32)
    m = jnp.max(logits, axis=1, keepdims=True)
    expl = jnp.exp(logits - m)
    gvc = 1.0 / jnp.sum(expl, axis=1, keepdims=True)
    top1 = jnp.argmax(logits, axis=1).astype(jnp.int32)
    onehot = (lax.broadcasted_iota(jnp.int32, logits.shape, 1)
              == top1[:, None])
    counts = onehot.astype(jnp.int32)
    off = 1
    while off < counts.shape[0]:
        shifted = jnp.concatenate(
            [jnp.zeros((off, counts.shape[1]), counts.dtype), counts[:-off]],
            axis=0)
        counts = counts + shifted
        off *= 2
    rank = jnp.sum(jnp.where(onehot, counts - 1, 0), axis=1)
    kept = rank < cap
    dest = jnp.where(kept, top1 * cap + rank, -1)
    gv_ref[...] = jnp.sum(gvc, axis=1)
    dest_ref[...] = dest
    y_ref[pl.ds(0, _T), :] = x * gvc
    y_ref[pl.ds(_T, 8), :] = jnp.zeros((8, _D), jnp.float32)


def _slotmap_body(dest_ref, gidx_ref, nslot):
    def init(s, _):
        gidx_ref[s] = _T
        return 0

    lax.fori_loop(0, nslot, init, 0)

    def fill(t, _):
        d = dest_ref[t]

        @pl.when(d >= 0)
        def _():
            gidx_ref[d] = t

        return 0

    lax.fori_loop(0, _T, fill, 0)


def _sc_dispatch_body(gidx_hbm, y_hbm, out_hbm,
                      idx_v, rows_v, sem, rows_per_w, chunk):
    nc = 2
    wid = lax.axis_index("s") * nc + lax.axis_index("c")
    base = wid * rows_per_w
    pltpu.sync_copy(gidx_hbm.at[pl.ds(base, rows_per_w)], idx_v)
    for part in range(rows_per_w // chunk):
        off = part * chunk
        pltpu.async_copy(
            y_hbm.at[idx_v.at[pl.ds(off, chunk)]], rows_v, sem).wait()
        pltpu.sync_copy(rows_v, out_hbm.at[pl.ds(base + off, chunk)])


def _combine_body(code_ref, gv_ref, comb_ref, sec_ref, tb, cap):
    i = pl.program_id(0)
    pos_e = lax.broadcasted_iota(jnp.int32, (_E, cap), 0)
    pos_c = lax.broadcasted_iota(jnp.int32, (_E, cap), 1)
    pos = pos_e * cap + pos_c

    def body(t, _):
        code_t = code_ref[i * tb + t]
        gv_t = gv_ref[i * tb + t]
        hit = pos == code_t
        comb_ref[pl.ds(t, 1)] = jnp.where(hit, gv_t, 0.0)[None]
        sec_ref[pl.ds(t, 1)] = hit[None]
        return 0

    lax.fori_loop(0, tb, body, 0)


def kernel(x, gate_weight):
    cap = _capacity(_T, _E)
    ec = _E * cap

    router = pl.pallas_call(
        functools.partial(_router_body, cap=cap),
        out_shape=(
            jax.ShapeDtypeStruct((_T,), jnp.float32),
            jax.ShapeDtypeStruct((_T,), jnp.int32),
            jax.ShapeDtypeStruct((_T + 8, _D), jnp.float32),
        ),
    )
    gv, dest, y = router(x, gate_weight)

    slotmap = pl.pallas_call(
        functools.partial(_slotmap_body, nslot=ec),
        in_specs=[pl.BlockSpec(memory_space=pltpu.SMEM)],
        out_specs=pl.BlockSpec(memory_space=pltpu.SMEM),
        out_shape=jax.ShapeDtypeStruct((ec,), jnp.int32),
    )
    gidx = slotmap(dest)

    nw = 32
    rows_per_w = ec // nw
    chunk = 80
    mesh = plsc.VectorSubcoreMesh(core_axis_name="c", subcore_axis_name="s")
    sc_dispatch = functools.partial(
        pl.kernel,
        mesh=mesh,
        out_type=jax.ShapeDtypeStruct((ec, _D), jnp.float32),
        scratch_types=[
            pltpu.VMEM((rows_per_w,), jnp.int32),
            pltpu.VMEM((chunk, _D), jnp.float32),
            pltpu.SemaphoreType.DMA,
        ],
    )(functools.partial(_sc_dispatch_body,
                        rows_per_w=rows_per_w, chunk=chunk))
    dispatched = sc_dispatch(gidx, y).reshape(_E, cap, _D)

    tb = 256
    nblk = _T // tb
    smem_spec = pl.BlockSpec(memory_space=pltpu.SMEM)
    combine = pl.pallas_call(
        functools.partial(_combine_body, tb=tb, cap=cap),
        grid=(nblk,),
        in_specs=[smem_spec, smem_spec],
        out_specs=(
            pl.BlockSpec((tb, _E, cap), lambda i: (i, 0, 0)),
            pl.BlockSpec((tb, _E, cap), lambda i: (i, 0, 0)),
        ),
        out_shape=(
            jax.ShapeDtypeStruct((_T, _E, cap), jnp.float32),
            jax.ShapeDtypeStruct((_T, _E, cap), jnp.bool_),
        ),
    )
    comb, sec = combine(dest, gv)
    return dispatched, comb, sec
